# baseline (device time: 175810 ns/iter reference)
import jax
import jax.numpy as jnp
from jax import lax
from jax.experimental import pallas as pl
from jax.experimental.pallas import tpu as pltpu

N_DEV = 4


def kernel(x, w_mat, scale_x, scale_w):
    m_per, k = x.shape
    n_per = w_mat.shape[1]

    x8 = x.astype(jnp.float8_e4m3fn)
    wb = w_mat.astype(jnp.bfloat16)
    scale = (scale_x.astype(jnp.float32) * scale_w.astype(jnp.float32)).reshape(1)

    def body(x_ref, w_ref, scale_ref, out_ref, comm_ref, send_sems, recv_sems):
        my = lax.axis_index("i")
        left = lax.rem(my + N_DEV - 1, N_DEV)
        right = lax.rem(my + 1, N_DEV)

        barrier_sem = pltpu.get_barrier_semaphore()
        for nbr in (left, right):
            pl.semaphore_signal(
                barrier_sem, inc=1,
                device_id=(nbr,), device_id_type=pl.DeviceIdType.MESH,
            )
        pl.semaphore_wait(barrier_sem, 2)

        comm_ref[0] = x_ref[...]
        s = scale_ref[0]

        def compute(chunk_fp8, origin):
            xb = chunk_fp8.astype(jnp.bfloat16)
            acc = jnp.dot(xb, w_ref[...], preferred_element_type=jnp.float32)
            y = acc * s
            z = jnp.clip(y, -60.0, 60.0)
            out_ref[pl.ds(origin * m_per, m_per), :] = y / (1.0 + jnp.exp(-z))

        compute(x_ref[...], my)

        for h in range(N_DEV - 1):
            send_slot = h % 2
            recv_slot = (h + 1) % 2
            rdma = pltpu.make_async_remote_copy(
                src_ref=comm_ref.at[send_slot],
                dst_ref=comm_ref.at[recv_slot],
                send_sem=send_sems.at[send_slot],
                recv_sem=recv_sems.at[recv_slot],
                device_id=(right,),
                device_id_type=pl.DeviceIdType.MESH,
            )
            rdma.start()
            rdma.wait()
            origin = lax.rem(my - (h + 1) + N_DEV, N_DEV)
            compute(comm_ref[recv_slot], origin)

    return pl.pallas_call(
        body,
        out_shape=jax.ShapeDtypeStruct((N_DEV * m_per, n_per), jnp.float32),
        in_specs=[
            pl.BlockSpec(memory_space=pltpu.VMEM),
            pl.BlockSpec(memory_space=pltpu.VMEM),
            pl.BlockSpec(memory_space=pltpu.SMEM),
        ],
        out_specs=pl.BlockSpec(memory_space=pltpu.VMEM),
        scratch_shapes=[
            pltpu.VMEM((2, m_per, k), jnp.float8_e4m3fn),
            pltpu.SemaphoreType.DMA((2,)),
            pltpu.SemaphoreType.DMA((2,)),
        ],
        compiler_params=pltpu.CompilerParams(collective_id=0),
    )(x8, wb, scale)


# device time: 92886 ns/iter; 1.8928x vs baseline; 1.8928x over previous
import jax
import jax.numpy as jnp
from jax import lax
from jax.experimental import pallas as pl
from jax.experimental.pallas import tpu as pltpu

N_DEV = 4
N_HOP = N_DEV - 1


def kernel(x, w_mat, scale_x, scale_w):
    m_per, k = x.shape
    n_per = w_mat.shape[1]
    half = m_per // 2

    x8 = x.astype(jnp.float8_e4m3fn)
    wb = w_mat.astype(jnp.bfloat16)
    scale = (scale_x.astype(jnp.float32) * scale_w.astype(jnp.float32)).reshape(1)

    def body(x_ref, w_ref, scale_ref, out_ref,
             cw_ref, ccw_ref, cw_send, cw_recv, ccw_send, ccw_recv):
        my = lax.axis_index("i")
        left = lax.rem(my + N_DEV - 1, N_DEV)
        right = lax.rem(my + 1, N_DEV)

        barrier_sem = pltpu.get_barrier_semaphore()
        for nbr in (left, right):
            pl.semaphore_signal(
                barrier_sem, inc=1,
                device_id=(nbr,), device_id_type=pl.DeviceIdType.MESH,
            )
        pl.semaphore_wait(barrier_sem, 2)

        def hop(h):
            cw_src = x_ref.at[pl.ds(0, half)] if h == 0 else cw_ref.at[h - 1]
            ccw_src = x_ref.at[pl.ds(half, half)] if h == 0 else ccw_ref.at[h - 1]
            cw = pltpu.make_async_remote_copy(
                src_ref=cw_src, dst_ref=cw_ref.at[h],
                send_sem=cw_send.at[h], recv_sem=cw_recv.at[h],
                device_id=(right,), device_id_type=pl.DeviceIdType.MESH,
            )
            ccw = pltpu.make_async_remote_copy(
                src_ref=ccw_src, dst_ref=ccw_ref.at[h],
                send_sem=ccw_send.at[h], recv_sem=ccw_recv.at[h],
                device_id=(left,), device_id_type=pl.DeviceIdType.MESH,
            )
            return cw, ccw

        s = scale_ref[0]

        def silu_store(chunk_fp8, row0, rows):
            xb = chunk_fp8.astype(jnp.bfloat16)
            acc = jnp.dot(xb, w_ref[...], preferred_element_type=jnp.float32)
            y = acc * s
            z = jnp.clip(y, -60.0, 60.0)
            out_ref[pl.ds(row0, rows), :] = y / (1.0 + jnp.exp(-z))

        rdmas = []
        cw0, ccw0 = hop(0)
        cw0.start()
        ccw0.start()
        rdmas.append((cw0, ccw0))

        silu_store(x_ref[...], my * m_per, m_per)

        for h in range(1, N_HOP):
            cw_p, ccw_p = rdmas[h - 1]
            cw_p.wait_recv()
            ccw_p.wait_recv()
            cw_h, ccw_h = hop(h)
            cw_h.start()
            ccw_h.start()
            rdmas.append((cw_h, ccw_h))
            top_origin = lax.rem(my - h + N_DEV, N_DEV)
            bot_origin = lax.rem(my + h, N_DEV)
            silu_store(cw_ref[h - 1], top_origin * m_per, half)
            silu_store(ccw_ref[h - 1], bot_origin * m_per + half, half)

        cw_l, ccw_l = rdmas[-1]
        cw_l.wait_recv()
        ccw_l.wait_recv()
        silu_store(cw_ref[N_HOP - 1], right * m_per, half)
        silu_store(ccw_ref[N_HOP - 1], left * m_per + half, half)

        for cw_h, ccw_h in rdmas:
            cw_h.wait_send()
            ccw_h.wait_send()

    return pl.pallas_call(
        body,
        out_shape=jax.ShapeDtypeStruct((N_DEV * m_per, n_per), jnp.float32),
        in_specs=[
            pl.BlockSpec(memory_space=pltpu.VMEM),
            pl.BlockSpec(memory_space=pltpu.VMEM),
            pl.BlockSpec(memory_space=pltpu.SMEM),
        ],
        out_specs=pl.BlockSpec(memory_space=pltpu.VMEM),
        scratch_shapes=[
            pltpu.VMEM((N_HOP, half, k), jnp.float8_e4m3fn),
            pltpu.VMEM((N_HOP, half, k), jnp.float8_e4m3fn),
            pltpu.SemaphoreType.DMA((N_HOP,)),
            pltpu.SemaphoreType.DMA((N_HOP,)),
            pltpu.SemaphoreType.DMA((N_HOP,)),
            pltpu.SemaphoreType.DMA((N_HOP,)),
        ],
        compiler_params=pltpu.CompilerParams(collective_id=0),
    )(x8, wb, scale)
